# baseline (speedup 1.0000x reference)
import functools

import jax
import jax.numpy as jnp
from jax import lax
from jax.experimental import pallas as pl
from jax.experimental.pallas import tpu as pltpu
from jax.experimental.pallas import tpu_sc as plsc

_B = 16384
_DU = 64
_NI = 8
_DI = 8
_CHUNK = 128


def _sc_gather_pairs(table2, idx2d, n_chunks):
    b_per_w = n_chunks * _CHUNK
    mesh = plsc.VectorSubcoreMesh(core_axis_name="c", subcore_axis_name="s")

    @functools.partial(
        pl.kernel,
        mesh=mesh,
        out_type=jax.ShapeDtypeStruct((_B, 2 * _DU), jnp.float32),
        scratch_types=[
            pltpu.VMEM((n_chunks, _CHUNK), jnp.int32),
            pltpu.VMEM((b_per_w, 2 * _DU), jnp.float32),
            pltpu.SemaphoreType.DMA,
        ],
    )
    def gather_kernel(table_hbm, idx_hbm, out_hbm, idx_v, rows_v, sem):
        wid = lax.axis_index("s") * 2 + lax.axis_index("c")
        pltpu.sync_copy(idx_hbm.at[pl.ds(wid * n_chunks, n_chunks)], idx_v)
        copies = []
        for j in range(n_chunks):
            copies.append(
                pltpu.async_copy(
                    table_hbm.at[idx_v.at[j]],
                    rows_v.at[pl.ds(j * _CHUNK, _CHUNK)],
                    sem,
                )
            )
        for c in copies:
            c.wait()
        pltpu.sync_copy(rows_v, out_hbm.at[pl.ds(wid * b_per_w, b_per_w)])

    return gather_kernel(table2, idx2d)


def _tc_mlp(g, parity, items_col, item_table, w1u_t, w1i_t, b1, w2_t, b2,
            w3_t, b3, w4_t, b4, blk):
    nb = _B // blk

    def body(g_ref, p_ref, it_ref, itab_ref, w1u_ref, w1i_ref, b1_ref,
             w2_ref, b2_ref, w3_ref, b3_ref, w4_ref, b4_ref, out_ref):
        gp = g_ref[:]
        par = p_ref[:] == 1
        x = jnp.where(par, gp[:, _DU:], gp[:, :_DU])
        it = it_ref[:]
        oh = (it == lax.broadcasted_iota(jnp.int32, (blk, _NI), 1))
        oh = oh.astype(jnp.float32)
        q = jnp.dot(itab_ref[:], w1i_ref[:],
                    preferred_element_type=jnp.float32)
        h = jnp.dot(x, w1u_ref[:], preferred_element_type=jnp.float32)
        h = h + jnp.dot(oh, q, preferred_element_type=jnp.float32) + b1_ref[:]
        h = jnp.maximum(h, 0.0)
        h = jnp.dot(h, w2_ref[:], preferred_element_type=jnp.float32) + b2_ref[:]
        h = jnp.maximum(h, 0.0)
        h = jnp.dot(h, w3_ref[:], preferred_element_type=jnp.float32) + b3_ref[:]
        h = jnp.maximum(h, 0.0)
        out_ref[:] = (jnp.dot(h, w4_ref[:],
                              preferred_element_type=jnp.float32) + b4_ref[:])

    full = lambda shape: pl.BlockSpec(shape, lambda i: (0,) * len(shape))
    return pl.pallas_call(
        body,
        grid=(nb,),
        in_specs=[
            pl.BlockSpec((blk, 2 * _DU), lambda i: (i, 0)),
            pl.BlockSpec((blk, 1), lambda i: (i, 0)),
            pl.BlockSpec((blk, 1), lambda i: (i, 0)),
            full((_NI, _DI)),
            full((_DU, 128)),
            full((_DI, 128)),
            full((1, 128)),
            full((128, 64)),
            full((1, 64)),
            full((64, 32)),
            full((1, 32)),
            full((32, 1)),
            full((1, 1)),
        ],
        out_specs=pl.BlockSpec((blk, 1), lambda i: (i, 0)),
        out_shape=jax.ShapeDtypeStruct((_B, 1), jnp.float32),
    )(g, parity, items_col, item_table, w1u_t, w1i_t, b1, w2_t, b2, w3_t,
      b3, w4_t, b4)


def kernel(users, items, user_table, item_table, W1, b1, W2, b2, W3, b3,
           W4, b4):
    users = users.astype(jnp.int32)
    items = items.astype(jnp.int32)

    info = plsc.get_sparse_core_info()
    n_workers = info.num_cores * info.num_subcores
    n_chunks = _B // (n_workers * _CHUNK)

    table2 = user_table.reshape(-1, 2 * _DU)
    idx2d = (users >> 1).reshape(n_workers * n_chunks, _CHUNK)
    g = _sc_gather_pairs(table2, idx2d, n_chunks)

    out2d = _tc_mlp(
        g,
        (users & 1).reshape(_B, 1),
        items.reshape(_B, 1),
        item_table,
        W1[:, :_DU].T,
        W1[:, _DU:].T,
        b1.reshape(1, -1),
        W2.T, b2.reshape(1, -1),
        W3.T, b3.reshape(1, -1),
        W4.T, b4.reshape(1, -1),
        blk=2048,
    )
    return out2d.reshape(_B)


# speedup vs baseline: 1.6762x; 1.6762x over previous
import functools

import jax
import jax.numpy as jnp
from jax import lax
from jax.experimental import pallas as pl
from jax.experimental.pallas import tpu as pltpu
from jax.experimental.pallas import tpu_sc as plsc

_B = 16384
_DU = 64
_NI = 8
_DI = 8
_WIN = 16


def _sc_gather(table, idx, n_workers):
    b_per_w = _B // n_workers
    mesh = plsc.VectorSubcoreMesh(core_axis_name="c", subcore_axis_name="s")

    @functools.partial(
        pl.kernel,
        mesh=mesh,
        out_type=jax.ShapeDtypeStruct((_B, _DU), jnp.float32),
        scratch_types=[
            pltpu.VMEM((b_per_w,), jnp.int32),
            pltpu.VMEM((b_per_w, _DU), jnp.float32),
            pltpu.SemaphoreType.DMA,
        ],
    )
    def gather_kernel(table_hbm, idx_hbm, out_hbm, idx_v, rows_v, sem):
        wid = lax.axis_index("s") * 2 + lax.axis_index("c")
        base = wid * b_per_w
        pltpu.sync_copy(idx_hbm.at[pl.ds(base, b_per_w)], idx_v)

        n_grp = b_per_w // 16

        def issue(g, _):
            v = idx_v[pl.ds(g * 16, 16)]
            for k in range(16):
                pltpu.make_async_copy(
                    table_hbm.at[pl.ds(v[k], 1)],
                    rows_v.at[pl.ds(g * 16 + k, 1)],
                    sem,
                ).start()

            @pl.when(g >= 1)
            def _drain():
                for k in range(16):
                    pltpu.make_async_copy(
                        table_hbm.at[pl.ds(0, 1)],
                        rows_v.at[pl.ds((g - 1) * 16 + k, 1)],
                        sem,
                    ).wait()

            return 0

        lax.fori_loop(0, n_grp, issue, 0)
        for k in range(16):
            pltpu.make_async_copy(
                table_hbm.at[pl.ds(0, 1)],
                rows_v.at[pl.ds((n_grp - 1) * 16 + k, 1)],
                sem,
            ).wait()
        pltpu.sync_copy(rows_v, out_hbm.at[pl.ds(base, b_per_w)])

    return gather_kernel(table, idx)


def _tc_mlp(u, items_col, item_table, w1u_t, w1i_t, b1, w2_t, b2, w3_t, b3,
            w4_t, b4, blk):
    nb = _B // blk

    def body(u_ref, it_ref, itab_ref, w1u_ref, w1i_ref, b1_ref, w2_ref,
             b2_ref, w3_ref, b3_ref, w4_ref, b4_ref, out_ref):
        x = u_ref[:]
        it = it_ref[:]
        oh = (it == lax.broadcasted_iota(jnp.int32, (blk, _NI), 1))
        oh = oh.astype(jnp.float32)
        q = jnp.dot(itab_ref[:], w1i_ref[:],
                    preferred_element_type=jnp.float32)
        h = jnp.dot(x, w1u_ref[:], preferred_element_type=jnp.float32)
        h = h + jnp.dot(oh, q, preferred_element_type=jnp.float32) + b1_ref[:]
        h = jnp.maximum(h, 0.0)
        h = jnp.dot(h, w2_ref[:], preferred_element_type=jnp.float32) + b2_ref[:]
        h = jnp.maximum(h, 0.0)
        h = jnp.dot(h, w3_ref[:], preferred_element_type=jnp.float32) + b3_ref[:]
        h = jnp.maximum(h, 0.0)
        out_ref[:] = (jnp.dot(h, w4_ref[:],
                              preferred_element_type=jnp.float32) + b4_ref[:])

    full = lambda shape: pl.BlockSpec(shape, lambda i: (0,) * len(shape))
    return pl.pallas_call(
        body,
        grid=(nb,),
        in_specs=[
            pl.BlockSpec((blk, _DU), lambda i: (i, 0)),
            pl.BlockSpec((blk, 1), lambda i: (i, 0)),
            full((_NI, _DI)),
            full((_DU, 128)),
            full((_DI, 128)),
            full((1, 128)),
            full((128, 64)),
            full((1, 64)),
            full((64, 32)),
            full((1, 32)),
            full((32, 1)),
            full((1, 1)),
        ],
        out_specs=pl.BlockSpec((blk, 1), lambda i: (i, 0)),
        out_shape=jax.ShapeDtypeStruct((_B, 1), jnp.float32),
    )(u, items_col, item_table, w1u_t, w1i_t, b1, w2_t, b2, w3_t, b3, w4_t,
      b4)


def kernel(users, items, user_table, item_table, W1, b1, W2, b2, W3, b3,
           W4, b4):
    users = users.astype(jnp.int32)
    items = items.astype(jnp.int32)

    info = plsc.get_sparse_core_info()
    n_workers = info.num_cores * info.num_subcores

    u = _sc_gather(user_table, users, n_workers)

    out2d = _tc_mlp(
        u,
        items.reshape(_B, 1),
        item_table,
        W1[:, :_DU].T,
        W1[:, _DU:].T,
        b1.reshape(1, -1),
        W2.T, b2.reshape(1, -1),
        W3.T, b3.reshape(1, -1),
        W4.T, b4.reshape(1, -1),
        blk=2048,
    )
    return out2d.reshape(_B)
